# baseline (speedup 1.0000x reference)
import functools

import jax
import jax.numpy as jnp
from jax import lax
from jax.experimental import pallas as pl
from jax.experimental.pallas import tpu as pltpu
from jax.experimental.pallas import tpu_sc as plsc

NUM_FACTORS = 64
BATCH = 16384
NC = 2
NS = 16
NW = NC * NS
B_PER_W = BATCH // NW
N_CHUNK = 4
C_ROWS = B_PER_W // N_CHUNK
G_ROWS = 16


def _gmf_body(users_h, items_h, utab_h, itab_h, wb_h,
              out_h,
              uidx, iidx, urows, irows, outv, wbv, tbuf, sem):
    wid = lax.axis_index("s") * NC + lax.axis_index("c")
    base = wid * B_PER_W

    pltpu.sync_copy(users_h.at[pl.ds(base, B_PER_W)], uidx)
    pltpu.sync_copy(items_h.at[pl.ds(base, B_PER_W)], iidx)
    pltpu.sync_copy(wb_h, wbv)

    copies = []
    for j in range(N_CHUNK):
        sl = pl.ds(j * C_ROWS, C_ROWS)
        copies.append(pltpu.async_copy(utab_h.at[uidx.at[sl]], urows.at[sl], sem))
        copies.append(pltpu.async_copy(itab_h.at[iidx.at[sl]], irows.at[sl], sem))
    for c in copies:
        c.wait()

    iota = lax.iota(jnp.int32, G_ROWS)
    wvecs = [wbv[pl.ds(16 * j, 16)] for j in range(5)]
    bias = wvecs[4][0]
    nsub = NUM_FACTORS // 16

    def group(g, _):
        for rr in range(G_ROWS):
            r = g * G_ROWS + rr
            t = jnp.zeros((16,), jnp.float32)
            for j in range(nsub):
                uv = urows[r, pl.ds(16 * j, 16)]
                iv = irows[r, pl.ds(16 * j, 16)]
                t = t + (uv * iv) * wvecs[j]
            plsc.store_scatter(tbuf, [iota * G_ROWS + rr], t)
        acc = jnp.full((G_ROWS,), bias, jnp.float32)
        for j in range(G_ROWS):
            acc = acc + tbuf[pl.ds(16 * j, 16)]
        outv[pl.ds(g * G_ROWS, G_ROWS)] = acc
        return _

    lax.fori_loop(0, B_PER_W // G_ROWS, group, None)
    pltpu.sync_copy(outv, out_h.at[pl.ds(base, B_PER_W)])


_gmf_sc = functools.partial(
    pl.kernel,
    out_type=jax.ShapeDtypeStruct((BATCH,), jnp.float32),
    mesh=plsc.VectorSubcoreMesh(core_axis_name="c", subcore_axis_name="s",
                                num_cores=NC, num_subcores=NS),
    compiler_params=pltpu.CompilerParams(needs_layout_passes=False,
                                         use_tc_tiling_on_sc=False),
    scratch_types=[
        pltpu.VMEM((B_PER_W,), jnp.int32),
        pltpu.VMEM((B_PER_W,), jnp.int32),
        pltpu.VMEM((B_PER_W, NUM_FACTORS), jnp.float32),
        pltpu.VMEM((B_PER_W, NUM_FACTORS), jnp.float32),
        pltpu.VMEM((B_PER_W,), jnp.float32),
        pltpu.VMEM((80,), jnp.float32),
        pltpu.VMEM((G_ROWS * G_ROWS,), jnp.float32),
        pltpu.SemaphoreType.DMA,
    ],
)(_gmf_body)


def kernel(users, items, user_emb_table, item_emb_table, fc_w, fc_b):
    wb = jnp.pad(jnp.concatenate([fc_w.reshape(-1), fc_b.reshape(-1)]), (0, 15))
    return _gmf_sc(users.astype(jnp.int32), items.astype(jnp.int32),
                   user_emb_table, item_emb_table, wb)
